# baseline (speedup 1.0000x reference)
import jax
import jax.numpy as jnp
from jax import lax
from jax.experimental import pallas as pl

VOCAB = 1000
BLOCK_B = 32


def _onehot_block(x_ref, o_ref):
    xi = x_ref[...]
    iota = lax.broadcasted_iota(jnp.int32, (xi.shape[0], xi.shape[1], VOCAB), 2)
    o_ref[...] = (xi[:, :, None] == iota).astype(jnp.float32)


def kernel(x):
    B, S = x.shape
    grid = (B // BLOCK_B,)
    return pl.pallas_call(
        _onehot_block,
        grid=grid,
        in_specs=[pl.BlockSpec((BLOCK_B, S), lambda i: (i, 0))],
        out_specs=pl.BlockSpec((BLOCK_B, S, VOCAB), lambda i: (i, 0, 0)),
        out_shape=jax.ShapeDtypeStruct((B, S, VOCAB), jnp.float32),
    )(x.astype(jnp.int32))
